# Initial kernel scaffold; baseline (speedup 1.0000x reference)
import jax
import jax.numpy as jnp
from jax import lax
from jax.experimental import pallas as pl
from jax.experimental.pallas import tpu as pltpu
from jax.experimental.pallas import tpu_sc as plsc

EMB = 64
NC, NS = 2, 16
NW = NC * NS
CHUNK = 128


def _emb_body(idx_hbm, table_hbm, out_hbm, idx_v, rows_v, sem):
    wid = lax.axis_index("s") * NC + lax.axis_index("c")
    b_per_w = idx_hbm.shape[0] // NW
    base = wid * b_per_w
    n_chunks = b_per_w // CHUNK

    @pl.loop(0, n_chunks)
    def _chunk(i):
        off = base + i * CHUNK
        pltpu.sync_copy(idx_hbm.at[pl.ds(off, CHUNK)], idx_v)
        pltpu.async_copy(table_hbm.at[idx_v], rows_v, sem).wait()
        pltpu.sync_copy(rows_v, out_hbm.at[pl.ds(off, CHUNK)])


def kernel(x, W):
    rows, cols = x.shape
    b = rows * cols
    xf = x.reshape(b).astype(jnp.int32)
    mesh = plsc.VectorSubcoreMesh(
        core_axis_name="c", subcore_axis_name="s",
        num_cores=NC, num_subcores=NS,
    )
    out = pl.kernel(
        _emb_body,
        out_type=jax.ShapeDtypeStruct((b, EMB), jnp.float32),
        mesh=mesh,
        scratch_types=[
            pltpu.VMEM((CHUNK,), jnp.int32),
            pltpu.VMEM((CHUNK, EMB), jnp.float32),
            pltpu.SemaphoreType.DMA,
        ],
    )(xf, W)
    return out.reshape(rows, cols, EMB)


# speedup vs baseline: 3.1988x; 3.1988x over previous
import jax
import jax.numpy as jnp
from jax import lax
from jax.experimental import pallas as pl
from jax.experimental.pallas import tpu as pltpu
from jax.experimental.pallas import tpu_sc as plsc

EMB = 64
NC, NS = 2, 16
NW = NC * NS
CHUNK = 128


def _emb_body(idx_hbm, table_hbm, out_hbm, idx_v, rows_v, sem):
    wid = lax.axis_index("s") * NC + lax.axis_index("c")
    b_per_w = idx_hbm.shape[0] // NW
    base = wid * b_per_w
    n_chunks = b_per_w // CHUNK

    @pl.loop(0, n_chunks)
    def _chunk(i):
        off = base + i * CHUNK
        pltpu.sync_copy(idx_hbm.at[pl.ds(off, CHUNK)], idx_v)
        pltpu.async_copy(table_hbm.at[idx_v], rows_v, sem).wait()
        pltpu.sync_copy(rows_v, out_hbm.at[pl.ds(off, CHUNK)])


def kernel(x, W):
    rows, cols = x.shape
    b = rows * cols
    xf = x.reshape(b).astype(jnp.int32)
    mesh = plsc.VectorSubcoreMesh(
        core_axis_name="c", subcore_axis_name="s",
        num_cores=NC, num_subcores=NS,
    )
    out = pl.kernel(
        _emb_body,
        out_type=jax.ShapeDtypeStruct((b, EMB), jnp.float32),
        mesh=mesh,
        scratch_types=[
            pltpu.VMEM((CHUNK,), jnp.int32),
            pltpu.VMEM((CHUNK, EMB), jnp.float32),
            pltpu.SemaphoreType.DMA,
        ],
        compiler_params=pltpu.CompilerParams(use_tc_tiling_on_sc=False),
    )(xf, W)
    return out.reshape(rows, cols, EMB)


# speedup vs baseline: 4.2615x; 1.3322x over previous
import jax
import jax.numpy as jnp
from jax import lax
from jax.experimental import pallas as pl
from jax.experimental.pallas import tpu as pltpu
from jax.experimental.pallas import tpu_sc as plsc

EMB = 64
NC, NS = 2, 16
NW = NC * NS
CHUNK = 128
K = 4
G = K * CHUNK


def _emb_body(idx_hbm, table_hbm, out_hbm, idx_all, rows_v, gsem, ssem):
    wid = lax.axis_index("s") * NC + lax.axis_index("c")
    n_chunks_w = idx_hbm.shape[0] // NW
    b_per_w = n_chunks_w * CHUNK
    n_outer = b_per_w // G
    base = wid * b_per_w

    pltpu.sync_copy(idx_hbm.at[pl.ds(wid * n_chunks_w, n_chunks_w)], idx_all)

    def gather_and_store(i, s):
        off = base + i * G
        descs = [
            pltpu.async_copy(
                table_hbm.at[idx_all.at[i * K + j]],
                rows_v.at[s, pl.ds(j * CHUNK, CHUNK)],
                gsem,
            )
            for j in range(K)
        ]
        for d in descs:
            d.wait()
        pltpu.async_copy(rows_v.at[s], out_hbm.at[pl.ds(off, G)], ssem)

    def drain_one_store(s):
        pltpu.make_async_copy(
            rows_v.at[s], out_hbm.at[pl.ds(base, G)], ssem
        ).wait()

    gather_and_store(0, 0)
    gather_and_store(1, 1)

    @pl.loop(2, n_outer, step=2)
    def _pair(i):
        for s in range(2):
            drain_one_store(s)
            gather_and_store(i + s, s)

    drain_one_store(0)
    drain_one_store(1)


def kernel(x, W):
    rows, cols = x.shape
    b = rows * cols
    xf = x.reshape(b // CHUNK, CHUNK).astype(jnp.int32)
    mesh = plsc.VectorSubcoreMesh(
        core_axis_name="c", subcore_axis_name="s",
        num_cores=NC, num_subcores=NS,
    )
    n_chunks_w = (b // CHUNK) // NW
    out = pl.kernel(
        _emb_body,
        out_type=jax.ShapeDtypeStruct((b, EMB), jnp.float32),
        mesh=mesh,
        scratch_types=[
            pltpu.VMEM((n_chunks_w, CHUNK), jnp.int32),
            pltpu.VMEM((2, G, EMB), jnp.float32),
            pltpu.SemaphoreType.DMA,
            pltpu.SemaphoreType.DMA,
        ],
        compiler_params=pltpu.CompilerParams(use_tc_tiling_on_sc=False),
    )(xf, W)
    return out.reshape(rows, cols, EMB)
